# baseline (speedup 1.0000x reference)
import functools

import jax
import jax.numpy as jnp
from jax import lax
from jax.experimental import pallas as pl
from jax.experimental.pallas import tpu as pltpu
from jax.experimental.pallas import tpu_sc as plsc

_B, _N, _E, _D = 2, 10000, 160000, 128
_TILES = 16
_EPT = _E // _TILES
_CHUNK = 80
_NCHUNK = _EPT // _CHUNK
_RPT = 632


_S = 3
_A = 2


def _sc_scatter_sum(msgs_flat, tgt_4d, zeros):
    mesh = plsc.VectorSubcoreMesh(core_axis_name="c", subcore_axis_name="s")

    @functools.partial(
        pl.kernel,
        out_type=jax.ShapeDtypeStruct((_B * _N, _D), jnp.float32),
        mesh=mesh,
        scratch_types=[
            pltpu.VMEM((_NCHUNK, _CHUNK), jnp.int32),
            pltpu.VMEM((_S, _CHUNK, _D), jnp.float32),
            pltpu.VMEM_SHARED((_N, _D), jnp.float32),
            pltpu.SemaphoreType.DMA,
            pltpu.SemaphoreType.DMA((_S,)),
            pltpu.SemaphoreType.DMA((_S,)),
        ],
    )
    def scatter_kernel(msg_hbm, tgt_hbm, zero_hbm, out_hbm,
                       idx_v, msg_v, acc_sh, isem, lsem, ssem):
        c = lax.axis_index("c")
        s = lax.axis_index("s")
        rbase = pl.multiple_of(jnp.minimum(s * _RPT, _N - _RPT), 8)
        idx_dma = pltpu.async_copy(tgt_hbm.at[c, s], idx_v, isem)
        pltpu.sync_copy(zero_hbm, acc_sh.at[pl.ds(rbase, _RPT)])
        ebase = c * _E + s * _EPT

        def issue_load(b, g):
            off = pl.multiple_of(ebase + g * _CHUNK, 8)
            pltpu.async_copy(msg_hbm.at[pl.ds(off, _CHUNK)], msg_v.at[b],
                             lsem.at[b])

        def wait_load(b):
            pltpu.make_async_copy(msg_hbm.at[pl.ds(0, _CHUNK)], msg_v.at[b],
                                  lsem.at[b]).wait()

        def fire_scatter(b, g):
            pltpu.async_copy(msg_v.at[b], acc_sh.at[idx_v.at[g]], ssem.at[b],
                             add=True)

        def wait_scatter(b):
            pltpu.make_async_copy(msg_v.at[b], acc_sh.at[idx_v.at[0]],
                                  ssem.at[b]).wait()

        for b in range(_A):
            issue_load(b, b)
        idx_dma.wait()
        plsc.subcore_barrier()

        def step(b, t, first=False):
            wait_load(b)
            fire_scatter(b, t)
            bn = (b + _A) % _S
            if not first:
                wait_scatter(bn)
            issue_load(bn, t + _A)

        for t in range(_S):
            step(t % _S, t, first=(t == 0))

        def body(g, carry):
            t0 = g * _S
            for b in range(_S):
                step(b, t0 + b)
            return carry

        n_steady = (_NCHUNK - _A) // _S - 1
        lax.fori_loop(1, 1 + n_steady, body, 0)
        for t in range(_S * (1 + n_steady), _NCHUNK):
            b = t % _S
            wait_load(b)
            fire_scatter(b, t)
            wait_scatter((b + _A) % _S)
        for t in range(_NCHUNK - _S + _A, _NCHUNK):
            wait_scatter(t % _S)
        plsc.subcore_barrier()
        pltpu.sync_copy(acc_sh.at[pl.ds(rbase, _RPT)],
                        out_hbm.at[pl.ds(c * _N + rbase, _RPT)])

    return scatter_kernel(msgs_flat, tgt_4d, zeros)


def _tc_gru(agg, h, W, U, b):
    M = agg.shape[0]
    blk = 10000

    def body(x_ref, h_ref, w_ref, u_ref, b_ref, o_ref):
        x = x_ref[...]
        hv = h_ref[...]
        xw = jnp.dot(x, w_ref[...], preferred_element_type=jnp.float32)
        hu = jnp.dot(hv, u_ref[...], preferred_element_type=jnp.float32)
        xw = xw + b_ref[0:1, :]
        hu = hu + b_ref[1:2, :]
        d = _D
        z = jax.nn.sigmoid(xw[:, :d] + hu[:, :d])
        r = jax.nn.sigmoid(xw[:, d:2 * d] + hu[:, d:2 * d])
        hh = jnp.tanh(xw[:, 2 * d:] + r * hu[:, 2 * d:])
        o_ref[...] = z * hv + (1.0 - z) * hh

    return pl.pallas_call(
        body,
        grid=(M // blk,),
        in_specs=[
            pl.BlockSpec((blk, _D), lambda i: (i, 0)),
            pl.BlockSpec((blk, _D), lambda i: (i, 0)),
            pl.BlockSpec((_D, 3 * _D), lambda i: (0, 0)),
            pl.BlockSpec((_D, 3 * _D), lambda i: (0, 0)),
            pl.BlockSpec((2, 3 * _D), lambda i: (0, 0)),
        ],
        out_specs=pl.BlockSpec((blk, _D), lambda i: (i, 0)),
        out_shape=jax.ShapeDtypeStruct((M, _D), jnp.float32),
    )(agg, h, W, U, b)


def kernel(atom_state, messages, connectivity, W, U, b):
    Bv, Nv, d = atom_state.shape
    tgt = connectivity[:, :, 1].reshape(_B, _TILES, _NCHUNK, _CHUNK)
    msgs = messages.reshape(-1, d)
    zeros = jnp.zeros((_RPT, d), jnp.float32)
    agg = _sc_scatter_sum(msgs, tgt, zeros)
    out = _tc_gru(agg, atom_state.reshape(-1, d), W, U, b)
    return out.reshape(Bv, Nv, d)


# speedup vs baseline: 1.0092x; 1.0092x over previous
import functools

import jax
import jax.numpy as jnp
from jax import lax
from jax.experimental import pallas as pl
from jax.experimental.pallas import tpu as pltpu
from jax.experimental.pallas import tpu_sc as plsc

_B, _N, _E, _D = 2, 10000, 160000, 128
_TILES = 16
_EPT = _E // _TILES
_CHUNK = 80
_NCHUNK = _EPT // _CHUNK
_RPT = 632


_S = 3
_A = 2


def _sc_scatter_sum(msgs_flat, tgt_4d, zeros):
    mesh = plsc.VectorSubcoreMesh(core_axis_name="c", subcore_axis_name="s")

    @functools.partial(
        pl.kernel,
        out_type=jax.ShapeDtypeStruct((_B * _N, _D), jnp.float32),
        mesh=mesh,
        scratch_types=[
            pltpu.VMEM((_NCHUNK, _CHUNK), jnp.int32),
            pltpu.VMEM((_S, _CHUNK, _D), jnp.float32),
            pltpu.VMEM_SHARED((_N, _D), jnp.float32),
            pltpu.SemaphoreType.DMA,
            pltpu.SemaphoreType.DMA((_S,)),
            pltpu.SemaphoreType.DMA((_S,)),
        ],
    )
    def scatter_kernel(msg_hbm, tgt_hbm, zero_hbm, out_hbm,
                       idx_v, msg_v, acc_sh, isem, lsem, ssem):
        c = lax.axis_index("c")
        s = lax.axis_index("s")
        rbase = pl.multiple_of(jnp.minimum(s * _RPT, _N - _RPT), 8)
        idx_dma = pltpu.async_copy(tgt_hbm.at[c, s], idx_v, isem)
        pltpu.sync_copy(zero_hbm, acc_sh.at[pl.ds(rbase, _RPT)])
        ebase = c * _E + s * _EPT

        def issue_load(b, g):
            off = pl.multiple_of(ebase + g * _CHUNK, 8)
            pltpu.async_copy(msg_hbm.at[pl.ds(off, _CHUNK)], msg_v.at[b],
                             lsem.at[b])

        def wait_load(b):
            pltpu.make_async_copy(msg_hbm.at[pl.ds(0, _CHUNK)], msg_v.at[b],
                                  lsem.at[b]).wait()

        def fire_scatter(b, g):
            pltpu.async_copy(msg_v.at[b], acc_sh.at[idx_v.at[g]], ssem.at[b],
                             add=True)

        def wait_scatter(b):
            pltpu.make_async_copy(msg_v.at[b], acc_sh.at[idx_v.at[0]],
                                  ssem.at[b]).wait()

        for b in range(_A):
            issue_load(b, b)
        idx_dma.wait()
        plsc.subcore_barrier()

        def step(b, t, first=False):
            wait_load(b)
            fire_scatter(b, t)
            bn = (b + _A) % _S
            if not first:
                wait_scatter(bn)
            issue_load(bn, t + _A)

        for t in range(_S):
            step(t % _S, t, first=(t == 0))

        def body(g, carry):
            t0 = g * _S
            for b in range(_S):
                step(b, t0 + b)
            return carry

        n_steady = (_NCHUNK - _A) // _S - 1
        lax.fori_loop(1, 1 + n_steady, body, 0)
        for t in range(_S * (1 + n_steady), _NCHUNK):
            b = t % _S
            wait_load(b)
            fire_scatter(b, t)
            wait_scatter((b + _A) % _S)
        for t in range(_NCHUNK - _S + _A, _NCHUNK):
            wait_scatter(t % _S)
        plsc.subcore_barrier()
        pltpu.sync_copy(acc_sh.at[pl.ds(rbase, _RPT)],
                        out_hbm.at[pl.ds(c * _N + rbase, _RPT)])

    return scatter_kernel(msgs_flat, tgt_4d, zeros)


def _tc_gru(agg, h, W, U, b):
    M = agg.shape[0]
    blk = 5000

    def body(x_ref, h_ref, w_ref, u_ref, b_ref, o_ref):
        x = x_ref[...]
        hv = h_ref[...]
        xw = jnp.dot(x, w_ref[...], preferred_element_type=jnp.float32)
        hu = jnp.dot(hv, u_ref[...], preferred_element_type=jnp.float32)
        xw = xw + b_ref[0:1, :]
        hu = hu + b_ref[1:2, :]
        d = _D
        z = jax.nn.sigmoid(xw[:, :d] + hu[:, :d])
        r = jax.nn.sigmoid(xw[:, d:2 * d] + hu[:, d:2 * d])
        hh = jnp.tanh(xw[:, 2 * d:] + r * hu[:, 2 * d:])
        o_ref[...] = z * hv + (1.0 - z) * hh

    return pl.pallas_call(
        body,
        grid=(M // blk,),
        in_specs=[
            pl.BlockSpec((blk, _D), lambda i: (i, 0)),
            pl.BlockSpec((blk, _D), lambda i: (i, 0)),
            pl.BlockSpec((_D, 3 * _D), lambda i: (0, 0)),
            pl.BlockSpec((_D, 3 * _D), lambda i: (0, 0)),
            pl.BlockSpec((2, 3 * _D), lambda i: (0, 0)),
        ],
        out_specs=pl.BlockSpec((blk, _D), lambda i: (i, 0)),
        out_shape=jax.ShapeDtypeStruct((M, _D), jnp.float32),
    )(agg, h, W, U, b)


def kernel(atom_state, messages, connectivity, W, U, b):
    Bv, Nv, d = atom_state.shape
    tgt = connectivity[:, :, 1].reshape(_B, _TILES, _NCHUNK, _CHUNK)
    msgs = messages.reshape(-1, d)
    zeros = jnp.zeros((_RPT, d), jnp.float32)
    agg = _sc_scatter_sum(msgs, tgt, zeros)
    out = _tc_gru(agg, atom_state.reshape(-1, d), W, U, b)
    return out.reshape(Bv, Nv, d)


# speedup vs baseline: 1.0148x; 1.0055x over previous
import functools

import jax
import jax.numpy as jnp
from jax import lax
from jax.experimental import pallas as pl
from jax.experimental.pallas import tpu as pltpu
from jax.experimental.pallas import tpu_sc as plsc

_B, _N, _E, _D = 2, 10000, 160000, 128
_TILES = 16
_EPT = _E // _TILES
_CHUNK = 80
_NCHUNK = _EPT // _CHUNK
_RPT = 632


_S = 3
_A = 2


def _sc_scatter_sum(msgs_flat, tgt_4d, zeros):
    mesh = plsc.VectorSubcoreMesh(core_axis_name="c", subcore_axis_name="s")

    @functools.partial(
        pl.kernel,
        out_type=jax.ShapeDtypeStruct((_B * _N, _D), jnp.float32),
        mesh=mesh,
        scratch_types=[
            pltpu.VMEM((_NCHUNK, _CHUNK), jnp.int32),
            pltpu.VMEM((_S, _CHUNK, _D), jnp.float32),
            pltpu.VMEM_SHARED((_N, _D), jnp.float32),
            pltpu.SemaphoreType.DMA,
            pltpu.SemaphoreType.DMA((_S,)),
            pltpu.SemaphoreType.DMA((_S,)),
        ],
    )
    def scatter_kernel(msg_hbm, tgt_hbm, zero_hbm, out_hbm,
                       idx_v, msg_v, acc_sh, isem, lsem, ssem):
        c = lax.axis_index("c")
        s = lax.axis_index("s")
        rbase = pl.multiple_of(jnp.minimum(s * _RPT, _N - _RPT), 8)
        idx_dma = pltpu.async_copy(tgt_hbm.at[c, s], idx_v, isem)
        pltpu.sync_copy(zero_hbm, acc_sh.at[pl.ds(rbase, _RPT)])
        ebase = c * _E + s * _EPT

        def issue_load(b, g):
            off = pl.multiple_of(ebase + g * _CHUNK, 8)
            pltpu.async_copy(msg_hbm.at[pl.ds(off, _CHUNK)], msg_v.at[b],
                             lsem.at[b])

        def wait_load(b):
            pltpu.make_async_copy(msg_hbm.at[pl.ds(0, _CHUNK)], msg_v.at[b],
                                  lsem.at[b]).wait()

        def fire_scatter(b, g):
            pltpu.async_copy(msg_v.at[b], acc_sh.at[idx_v.at[g]], ssem.at[b],
                             add=True)

        def wait_scatter(b):
            pltpu.make_async_copy(msg_v.at[b], acc_sh.at[idx_v.at[0]],
                                  ssem.at[b]).wait()

        for b in range(_A):
            issue_load(b, b)
        idx_dma.wait()
        plsc.subcore_barrier()

        def step(b, t, first=False):
            wait_load(b)
            fire_scatter(b, t)
            bn = (b + _A) % _S
            if not first:
                wait_scatter(bn)
            issue_load(bn, t + _A)

        for t in range(_S):
            step(t % _S, t, first=(t == 0))

        def body(g, carry):
            t0 = g * _S
            for b in range(_S):
                step(b, t0 + b)
            return carry

        n_steady = (_NCHUNK - _A) // _S - 1
        lax.fori_loop(1, 1 + n_steady, body, 0)
        for t in range(_S * (1 + n_steady), _NCHUNK):
            b = t % _S
            wait_load(b)
            fire_scatter(b, t)
            wait_scatter((b + _A) % _S)
        for t in range(_NCHUNK - _S + _A, _NCHUNK):
            wait_scatter(t % _S)
        plsc.subcore_barrier()
        pltpu.sync_copy(acc_sh.at[pl.ds(rbase, _RPT)],
                        out_hbm.at[pl.ds(c * _N + rbase, _RPT)])

    return scatter_kernel(msgs_flat, tgt_4d, zeros)


def _tc_gru(agg, h, W, U, b):
    M = agg.shape[0]
    blk = 4000

    def body(x_ref, h_ref, w_ref, u_ref, b_ref, o_ref):
        x = x_ref[...]
        hv = h_ref[...]
        xw = jnp.dot(x, w_ref[...], preferred_element_type=jnp.float32)
        hu = jnp.dot(hv, u_ref[...], preferred_element_type=jnp.float32)
        xw = xw + b_ref[0:1, :]
        hu = hu + b_ref[1:2, :]
        d = _D
        z = jax.nn.sigmoid(xw[:, :d] + hu[:, :d])
        r = jax.nn.sigmoid(xw[:, d:2 * d] + hu[:, d:2 * d])
        hh = jnp.tanh(xw[:, 2 * d:] + r * hu[:, 2 * d:])
        o_ref[...] = z * hv + (1.0 - z) * hh

    return pl.pallas_call(
        body,
        grid=(M // blk,),
        in_specs=[
            pl.BlockSpec((blk, _D), lambda i: (i, 0)),
            pl.BlockSpec((blk, _D), lambda i: (i, 0)),
            pl.BlockSpec((_D, 3 * _D), lambda i: (0, 0)),
            pl.BlockSpec((_D, 3 * _D), lambda i: (0, 0)),
            pl.BlockSpec((2, 3 * _D), lambda i: (0, 0)),
        ],
        out_specs=pl.BlockSpec((blk, _D), lambda i: (i, 0)),
        out_shape=jax.ShapeDtypeStruct((M, _D), jnp.float32),
    )(agg, h, W, U, b)


def kernel(atom_state, messages, connectivity, W, U, b):
    Bv, Nv, d = atom_state.shape
    tgt = connectivity[:, :, 1].reshape(_B, _TILES, _NCHUNK, _CHUNK)
    msgs = messages.reshape(-1, d)
    zeros = jnp.zeros((_RPT, d), jnp.float32)
    agg = _sc_scatter_sum(msgs, tgt, zeros)
    out = _tc_gru(agg, atom_state.reshape(-1, d), W, U, b)
    return out.reshape(Bv, Nv, d)
